# baseline (speedup 1.0000x reference)
import functools

import jax
import jax.numpy as jnp
from jax import lax
from jax.experimental import pallas as pl
from jax.experimental.pallas import tpu as pltpu
from jax.experimental.pallas import tpu_sc as plsc

_B = 16384
_L = 20
_D = 32
_N = _B * _L

_info = plsc.get_sparse_core_info()
_NC = _info.num_cores
_NS = _info.num_subcores
_NW = _NC * _NS
_RW = _B // _NW
_PER_W = _RW * _L
_RCH = 32
_TCH = _RCH * _L
_NCHUNK = _RW // _RCH
_NBUF = 4
_LANES = 16

_mesh = plsc.VectorSubcoreMesh(core_axis_name="c", subcore_axis_name="s")


@functools.partial(
    pl.kernel,
    mesh=_mesh,
    out_type=jax.ShapeDtypeStruct((_B, _L, _D), jnp.float32),
    scratch_types=[
        pltpu.VMEM((_RW, _L), jnp.int32),
        pltpu.VMEM((_PER_W,), jnp.int32),
        pltpu.VMEM((_NBUF, _TCH, _D), jnp.float32),
        pltpu.SemaphoreType.DMA((_NBUF,)),
        pltpu.SemaphoreType.DMA((_NBUF,)),
    ],
    compiler_params=pltpu.CompilerParams(
        use_tc_tiling_on_sc=False, needs_layout_passes=False),
)
def _gather(idx_hbm, table_hbm, out_hbm, idx2d, idx_flat, bufs, gsems, ssems):
    wid = lax.axis_index("s") * _NC + lax.axis_index("c")
    row0 = wid * _RW

    pltpu.sync_copy(idx_hbm.at[pl.ds(row0, _RW)], idx2d)

    def flatten_body(v, carry):
        base = v * _LANES
        p = base + jax.lax.iota(jnp.int32, _LANES)
        r = jax.lax.shift_right_logical(p * 3277, 16)
        c = p - r * _L
        vals = plsc.load_gather(idx2d, [r, c])
        idx_flat[pl.ds(base, _LANES)] = vals
        return carry

    lax.fori_loop(0, _PER_W // _LANES, flatten_body, 0)

    def start_gather(g):
        b = g % _NBUF
        return pltpu.async_copy(
            table_hbm.at[idx_flat.at[pl.ds(g * _TCH, _TCH)]],
            bufs.at[b],
            gsems.at[b])

    def issue_stores(g):
        b = g % _NBUF
        rowbase = row0 + g * _RCH

        def body(r, carry):
            pltpu.async_copy(
                bufs.at[b, pl.ds(r * _L, _L)],
                out_hbm.at[rowbase + r],
                ssems.at[b])
            return carry

        lax.fori_loop(0, _RCH, body, 0)

    def drain_stores(g):
        b = g % _NBUF

        def body(r, carry):
            pltpu.make_async_copy(
                bufs.at[b, pl.ds(0, _L)], out_hbm.at[row0], ssems.at[b]
            ).wait()
            return carry

        lax.fori_loop(0, _RCH, body, 0)

    gcopies = [None] * _NCHUNK
    drained = [False] * _NCHUNK
    for g in range(min(_NBUF, _NCHUNK)):
        gcopies[g] = start_gather(g)
    for g in range(_NCHUNK):
        ng = g - 1 + _NBUF
        if g >= 1 and ng < _NCHUNK:
            drain_stores(g - 1)
            drained[g - 1] = True
            gcopies[ng] = start_gather(ng)
        gcopies[g].wait()
        issue_stores(g)
    for g in range(_NCHUNK):
        if not drained[g]:
            drain_stores(g)


def kernel(token_id, weight):
    return _gather(token_id, weight)


# speedup vs baseline: 1.0586x; 1.0586x over previous
import functools

import jax
import jax.numpy as jnp
from jax import lax
from jax.experimental import pallas as pl
from jax.experimental.pallas import tpu as pltpu
from jax.experimental.pallas import tpu_sc as plsc

_B = 16384
_L = 20
_D = 32
_N = _B * _L

_info = plsc.get_sparse_core_info()
_NC = _info.num_cores
_NS = _info.num_subcores
_NW = _NC * _NS
_RW = _B // _NW
_PER_W = _RW * _L
_NBUF = 4
_LANES = 16
_VPC = _RW // _LANES

_mesh = plsc.VectorSubcoreMesh(core_axis_name="c", subcore_axis_name="s")


@functools.partial(
    pl.kernel,
    mesh=_mesh,
    out_type=jax.ShapeDtypeStruct((_L, _B, _D), jnp.float32),
    scratch_types=[
        pltpu.VMEM((_RW, _L), jnp.int32),
        pltpu.VMEM((_PER_W,), jnp.int32),
        pltpu.VMEM((_NBUF, _RW, _D), jnp.float32),
        pltpu.SemaphoreType.DMA((_NBUF,)),
        pltpu.SemaphoreType.DMA((_NBUF,)),
    ],
    compiler_params=pltpu.CompilerParams(
        use_tc_tiling_on_sc=False, needs_layout_passes=False),
)
def _gather(idx_hbm, table_hbm, out_hbm, idx2d, idx_flat, bufs, gsems, ssems):
    wid = lax.axis_index("s") * _NC + lax.axis_index("c")
    row0 = wid * _RW

    pltpu.sync_copy(idx_hbm.at[pl.ds(row0, _RW)], idx2d)

    def flatten_body(v, carry):
        c = v // _VPC
        r = (v % _VPC) * _LANES + jax.lax.iota(jnp.int32, _LANES)
        cvec = jnp.full((_LANES,), 0, jnp.int32) + c
        vals = plsc.load_gather(idx2d, [r, cvec])
        idx_flat[pl.ds(v * _LANES, _LANES)] = vals
        return carry

    lax.fori_loop(0, _L * _VPC, flatten_body, 0)

    def start_gather(g):
        b = g % _NBUF
        return pltpu.async_copy(
            table_hbm.at[idx_flat.at[pl.ds(g * _RW, _RW)]],
            bufs.at[b],
            gsems.at[b])

    def start_store(g):
        b = g % _NBUF
        return pltpu.async_copy(
            bufs.at[b], out_hbm.at[g, pl.ds(row0, _RW)], ssems.at[b])

    gcopies = [None] * _L
    scopies = [None] * _L
    for g in range(min(_NBUF, _L)):
        gcopies[g] = start_gather(g)
    for g in range(_L):
        ng = g - 1 + _NBUF
        if g >= 1 and ng < _L:
            scopies[g - 1].wait()
            gcopies[ng] = start_gather(ng)
        gcopies[g].wait()
        scopies[g] = start_store(g)
    for g in range(max(_L - _NBUF, 0), _L):
        if scopies[g] is not None:
            scopies[g].wait()


def kernel(token_id, weight):
    out_lbd = _gather(token_id, weight)
    return jnp.transpose(out_lbd, (1, 0, 2))
